# baseline (device time: 147188 ns/iter reference)
import jax
import jax.numpy as jnp
from jax import lax
from jax.experimental import pallas as pl
from jax.experimental.pallas import tpu as pltpu

M = 3072
N = 3072
K = 1536
M_HALF = M // 2
CHUNK_ROWS = [160] * 8 + [96, 96, 32, 32]
CHUNK_OFF = [sum(CHUNK_ROWS[:i]) for i in range(len(CHUNK_ROWS))]
N_CHUNKS = len(CHUNK_ROWS)
MAX_CHUNK = max(CHUNK_ROWS)
assert sum(CHUNK_ROWS) == M_HALF
B_QUARTER = K // 4


def kernel(A, B):
    def body(a_hbm, b_hbm, out_hbm, a_stage, b_stage, bb, out_vmem, recv_ref,
             a_sems, b_sems, copy_sems,
             send_sem1, recv_sem1, send_sem2, recv_sem2):
        my_x = lax.axis_index("x")
        my_y = lax.axis_index("y")
        x_nbr = (1 - my_x, my_y)
        y_nbr = (my_x, 1 - my_y)
        row0 = my_y * M_HALF

        def a_dma(c):
            return pltpu.make_async_copy(
                a_hbm.at[pl.ds(row0 + CHUNK_OFF[c], CHUNK_ROWS[c]), :],
                a_stage.at[c % 2, pl.ds(0, CHUNK_ROWS[c]), :],
                a_sems.at[c % 2],
            )

        def b_dma(q):
            return pltpu.make_async_copy(
                b_hbm.at[pl.ds(q * B_QUARTER, B_QUARTER), :],
                b_stage.at[q % 2],
                b_sems.at[q % 2],
            )

        def rdma1_chunk(c):
            sl = pl.ds(CHUNK_OFF[c], CHUNK_ROWS[c])
            return pltpu.make_async_remote_copy(
                src_ref=out_vmem.at[sl, :],
                dst_ref=recv_ref.at[sl, :],
                send_sem=send_sem1.at[c],
                recv_sem=recv_sem1.at[c],
                device_id=x_nbr,
                device_id_type=pl.DeviceIdType.MESH,
            )

        def rdma2_chunk(c):
            return pltpu.make_async_remote_copy(
                src_ref=out_vmem.at[pl.ds(CHUNK_OFF[c], CHUNK_ROWS[c]), :],
                dst_ref=out_hbm.at[pl.ds(row0 + CHUNK_OFF[c], CHUNK_ROWS[c]), :],
                send_sem=send_sem2.at[c],
                recv_sem=recv_sem2.at[c],
                device_id=y_nbr,
                device_id_type=pl.DeviceIdType.MESH,
            )

        def out_copy(c):
            return pltpu.make_async_copy(
                out_vmem.at[pl.ds(CHUNK_OFF[c], CHUNK_ROWS[c]), :],
                out_hbm.at[pl.ds(row0 + CHUNK_OFF[c], CHUNK_ROWS[c]), :],
                copy_sems.at[c],
            )

        b_dma(0).start()
        a_dma(0).start()
        a_dma(1).start()

        barrier = pltpu.get_barrier_semaphore()
        for nbr in (x_nbr, y_nbr):
            pl.semaphore_signal(
                barrier, inc=1,
                device_id=nbr, device_id_type=pl.DeviceIdType.MESH,
            )
        pl.semaphore_wait(barrier, 2)

        for q in range(4):
            if q + 1 < 4:
                b_dma(q + 1).start()
            b_dma(q).wait()
            bb[pl.ds(q * B_QUARTER, B_QUARTER), :] = (
                b_stage[q % 2].astype(jnp.bfloat16)
            )

        for c in range(N_CHUNKS):
            a_dma(c).wait()
            acc = jnp.dot(
                a_stage[c % 2, 0:CHUNK_ROWS[c], :].astype(jnp.bfloat16),
                bb[:, :],
                preferred_element_type=jnp.float32,
            )
            if c + 2 < N_CHUNKS:
                a_dma(c + 2).start()
            out_vmem[pl.ds(CHUNK_OFF[c], CHUNK_ROWS[c]), :] = (
                acc.astype(jnp.bfloat16)
            )
            rdma1_chunk(c).start()

        for c in range(N_CHUNKS):
            rdma1_chunk(c).wait()
            sl = pl.ds(CHUNK_OFF[c], CHUNK_ROWS[c])
            out_vmem[sl, :] = out_vmem[sl, :] + recv_ref[sl, :]
            rdma2_chunk(c).start()
            out_copy(c).start()

        for c in range(N_CHUNKS):
            rdma2_chunk(c).wait()
            out_copy(c).wait()

    return pl.pallas_call(
        body,
        out_shape=jax.ShapeDtypeStruct((M, N), jnp.bfloat16),
        in_specs=[
            pl.BlockSpec(memory_space=pl.ANY),
            pl.BlockSpec(memory_space=pl.ANY),
        ],
        out_specs=pl.BlockSpec(memory_space=pl.ANY),
        scratch_shapes=[
            pltpu.VMEM((2, MAX_CHUNK, K), jnp.float32),
            pltpu.VMEM((2, B_QUARTER, N), jnp.float32),
            pltpu.VMEM((K, N), jnp.bfloat16),
            pltpu.VMEM((M_HALF, N), jnp.bfloat16),
            pltpu.VMEM((M_HALF, N), jnp.bfloat16),
            pltpu.SemaphoreType.DMA((2,)),
            pltpu.SemaphoreType.DMA((2,)),
            pltpu.SemaphoreType.DMA((N_CHUNKS,)),
            pltpu.SemaphoreType.DMA((N_CHUNKS,)),
            pltpu.SemaphoreType.DMA((N_CHUNKS,)),
            pltpu.SemaphoreType.DMA((N_CHUNKS,)),
            pltpu.SemaphoreType.DMA((N_CHUNKS,)),
        ],
        compiler_params=pltpu.CompilerParams(
            collective_id=0,
            vmem_limit_bytes=64 * 1024 * 1024,
        ),
    )(A, B)


# device time: 142355 ns/iter; 1.0340x vs baseline; 1.0340x over previous
import jax
import jax.numpy as jnp
from jax import lax
from jax.experimental import pallas as pl
from jax.experimental.pallas import tpu as pltpu

M = 3072
N = 3072
K = 1536
M_HALF = M // 2
CHUNK_ROWS = [192] * 8
CHUNK_OFF = [sum(CHUNK_ROWS[:i]) for i in range(len(CHUNK_ROWS))]
N_CHUNKS = len(CHUNK_ROWS)
MAX_CHUNK = max(CHUNK_ROWS)
assert sum(CHUNK_ROWS) == M_HALF
B_QUARTER = K // 4


def kernel(A, B):
    def body(a_hbm, b_hbm, out_hbm, a_stage, b_stage, bb, out_vmem, recv_ref,
             a_sems, b_sems, copy_sems,
             send_sem1, recv_sem1, send_sem2, recv_sem2):
        my_x = lax.axis_index("x")
        my_y = lax.axis_index("y")
        x_nbr = (1 - my_x, my_y)
        y_nbr = (my_x, 1 - my_y)
        row0 = my_y * M_HALF

        def a_dma(c):
            return pltpu.make_async_copy(
                a_hbm.at[pl.ds(row0 + CHUNK_OFF[c], CHUNK_ROWS[c]), :],
                a_stage.at[c % 2, pl.ds(0, CHUNK_ROWS[c]), :],
                a_sems.at[c % 2],
            )

        def b_dma(q):
            return pltpu.make_async_copy(
                b_hbm.at[pl.ds(q * B_QUARTER, B_QUARTER), :],
                b_stage.at[q % 2],
                b_sems.at[q % 2],
            )

        def rdma1_chunk(c):
            sl = pl.ds(CHUNK_OFF[c], CHUNK_ROWS[c])
            return pltpu.make_async_remote_copy(
                src_ref=out_vmem.at[sl, :],
                dst_ref=recv_ref.at[sl, :],
                send_sem=send_sem1.at[c],
                recv_sem=recv_sem1.at[c],
                device_id=x_nbr,
                device_id_type=pl.DeviceIdType.MESH,
            )

        def rdma2_chunk(c):
            return pltpu.make_async_remote_copy(
                src_ref=recv_ref.at[pl.ds(CHUNK_OFF[c], CHUNK_ROWS[c]), :],
                dst_ref=out_hbm.at[pl.ds(row0 + CHUNK_OFF[c], CHUNK_ROWS[c]), :],
                send_sem=send_sem2.at[c],
                recv_sem=recv_sem2.at[c],
                device_id=y_nbr,
                device_id_type=pl.DeviceIdType.MESH,
            )

        def out_copy(c):
            return pltpu.make_async_copy(
                recv_ref.at[pl.ds(CHUNK_OFF[c], CHUNK_ROWS[c]), :],
                out_hbm.at[pl.ds(row0 + CHUNK_OFF[c], CHUNK_ROWS[c]), :],
                copy_sems.at[c],
            )

        b_dma(0).start()
        a_dma(0).start()
        a_dma(1).start()

        barrier = pltpu.get_barrier_semaphore()
        for nbr in (x_nbr, y_nbr):
            pl.semaphore_signal(
                barrier, inc=1,
                device_id=nbr, device_id_type=pl.DeviceIdType.MESH,
            )

        for q in range(4):
            if q + 1 < 4:
                b_dma(q + 1).start()
            b_dma(q).wait()
            bb[pl.ds(q * B_QUARTER, B_QUARTER), :] = (
                b_stage[q % 2].astype(jnp.bfloat16)
            )

        pl.semaphore_wait(barrier, 2)

        for c in range(N_CHUNKS):
            a_dma(c).wait()
            acc = jnp.dot(
                a_stage[c % 2, 0:CHUNK_ROWS[c], :].astype(jnp.bfloat16),
                bb[:, :],
                preferred_element_type=jnp.float32,
            )
            if c + 2 < N_CHUNKS:
                a_dma(c + 2).start()
            out_vmem[pl.ds(CHUNK_OFF[c], CHUNK_ROWS[c]), :] = (
                acc.astype(jnp.bfloat16)
            )
            rdma1_chunk(c).start()

        for c in range(N_CHUNKS):
            rdma1_chunk(c).wait_recv()
            sl = pl.ds(CHUNK_OFF[c], CHUNK_ROWS[c])
            recv_ref[sl, :] = out_vmem[sl, :] + recv_ref[sl, :]
            rdma2_chunk(c).start()
            out_copy(c).start()

        for c in range(N_CHUNKS):
            rdma1_chunk(c).wait_send()
            rdma2_chunk(c).wait()
            out_copy(c).wait()

    return pl.pallas_call(
        body,
        out_shape=jax.ShapeDtypeStruct((M, N), jnp.bfloat16),
        in_specs=[
            pl.BlockSpec(memory_space=pl.ANY),
            pl.BlockSpec(memory_space=pl.ANY),
        ],
        out_specs=pl.BlockSpec(memory_space=pl.ANY),
        scratch_shapes=[
            pltpu.VMEM((2, MAX_CHUNK, K), jnp.float32),
            pltpu.VMEM((2, B_QUARTER, N), jnp.float32),
            pltpu.VMEM((K, N), jnp.bfloat16),
            pltpu.VMEM((M_HALF, N), jnp.bfloat16),
            pltpu.VMEM((M_HALF, N), jnp.bfloat16),
            pltpu.SemaphoreType.DMA((2,)),
            pltpu.SemaphoreType.DMA((2,)),
            pltpu.SemaphoreType.DMA((N_CHUNKS,)),
            pltpu.SemaphoreType.DMA((N_CHUNKS,)),
            pltpu.SemaphoreType.DMA((N_CHUNKS,)),
            pltpu.SemaphoreType.DMA((N_CHUNKS,)),
            pltpu.SemaphoreType.DMA((N_CHUNKS,)),
        ],
        compiler_params=pltpu.CompilerParams(
            collective_id=0,
            vmem_limit_bytes=64 * 1024 * 1024,
        ),
    )(A, B)
